# baseline (speedup 1.0000x reference)
import functools

import jax
import jax.numpy as jnp
from jax import lax
from jax.experimental import pallas as pl
from jax.experimental.pallas import tpu as pltpu
from jax.experimental.pallas import tpu_sc as plsc

N = 1048576
B = 256
VPB = B // 16
NW = 32
PW = N // NW
NBLK = PW // B

F32 = jnp.float32
I32 = jnp.int32

CLIP_HI = 1.0 - 1e-6

R_RA = 0
R_LV = 2
R_NHL = 18
R_RAD = 34
R_LVR = 46
NSTREAM = 50


def _spl_i(v):
    return jnp.full((16,), v, I32)


def _rnd_even(u):
    f = u.astype(I32)
    r = u - f.astype(F32)
    odd = (f & 1) == 1
    up = (r > 0.5) | ((r == 0.5) & odd)
    return jnp.where(up, f + 1, f)


def _bilin_idx(x0, x1, res):
    u = jnp.clip(x0, 0.0, 1.0) * (res - 1)
    v = jnp.clip(x1, 0.0, 1.0) * (res - 1)
    u0i = u.astype(I32)
    v0i = v.astype(I32)
    u1i = jnp.minimum(u0i + 1, res - 1)
    v1i = jnp.minimum(v0i + 1, res - 1)
    fu = u - u0i.astype(F32)
    fv = v - v0i.astype(F32)
    c00 = u0i * res + v0i
    c01 = u0i * res + v1i
    c10 = u1i * res + v0i
    c11 = u1i * res + v1i
    return (c00, c01, c10, c11), fu, fv


def _sc_body(rough_h, aniso_h, ndlv_h, metal_h, subcg_h, spst_h, shst_h,
             lum_h, raenc_h, cge_h, rs_h, rad_h, cgd_h, lvr_h, lv_h, nhl_h,
             outx_h, outy_h,
             inb, idxb, g, fr, t_rs, t_cge, t_cgd, ox, oy,
             sem_in, sem_ra, sem_b):
    wid = lax.axis_index("s") * 2 + lax.axis_index("c")

    pltpu.sync_copy(rs_h, t_rs)
    pltpu.sync_copy(cge_h, t_cge)
    pltpu.sync_copy(cgd_h, t_cgd)

    inputs_h = (rough_h, aniso_h, ndlv_h, metal_h, subcg_h, spst_h, shst_h,
                lum_h)

    def col(k, lanes, c):
        return plsc.load_gather(inb, [lanes + k * B, _spl_i(c)])

    def body(blk, carry):
        base = wid * PW + blk * B

        cps = [pltpu.async_copy(inp.at[pl.ds(base, B)],
                                inb.at[pl.ds(k * B, B)], sem_in)
               for k, inp in enumerate(inputs_h)]
        for cp in cps:
            cp.wait()

        for i in range(VPB):
            sl = pl.ds(i * 16, 16)
            lanes = lax.iota(I32, 16) + i * 16
            r0 = jnp.clip(col(0, lanes, 0), 0.0, 1.0) * 2047.0
            a0 = jnp.clip(col(1, lanes, 0), 0.0, 1.0) * 2047.0
            e = (_rnd_even(r0) * 2048 + _rnd_even(a0)) * 2
            idxb[pl.ds(R_RA * B + i * 16, 16)] = e
            idxb[pl.ds((R_RA + 1) * B + i * 16, 16)] = e + 1
        cps_ra = [pltpu.async_copy(
            raenc_h.at[idxb.at[pl.ds(R_RA * B, 2 * B)]],
            g.at[pl.ds(R_RA * B, 2 * B)], sem_ra)]

        for i in range(VPB):
            sl = pl.ds(i * 16, 16)
            lanes = lax.iota(I32, 16) + i * 16
            nd0 = col(2, lanes, 0)
            nd1 = col(2, lanes, 1)
            cs, fu, fv = _bilin_idx(nd0, nd1, 1024)
            for c in range(4):
                e = cs[c] * 4
                for ch in range(4):
                    idxb[pl.ds((R_LV + 4 * c + ch) * B + i * 16, 16)] = e + ch
            fr[0, sl] = fu
            fr[1, sl] = fv
            m1 = col(3, lanes, 1)
            cs, fu, fv = _bilin_idx(m1, nd0, 1024)
            for c in range(4):
                e = cs[c] * 4
                for ch in range(4):
                    idxb[pl.ds((R_NHL + 4 * c + ch) * B + i * 16, 16)] = e + ch
            fr[2, sl] = fu
            fr[3, sl] = fv
        cps_b = [pltpu.async_copy(
            lv_h.at[idxb.at[pl.ds(R_LV * B, 16 * B)]],
            g.at[pl.ds(R_LV * B, 16 * B)], sem_b)]
        cps_b.append(pltpu.async_copy(
            nhl_h.at[idxb.at[pl.ds(R_NHL * B, 16 * B)]],
            g.at[pl.ds(R_NHL * B, 16 * B)], sem_b))

        for i in range(VPB):
            sl = pl.ds(i * 16, 16)
            lanes = lax.iota(I32, 16) + i * 16
            sc1 = jnp.clip(col(4, lanes, 1), 0.0, 1.0) * 2047.0
            cgv = plsc.load_gather(t_cge, [_rnd_even(sc1)])
            cg0 = jnp.clip(cgv, 0.0, CLIP_HI)
            u = jnp.clip(cg0, 0.0, 1.0) * 2047.0
            u0i = u.astype(I32)
            u1i = jnp.minimum(u0i + 1, 2047)
            fu = u - u0i.astype(F32)
            gfu = 1.0 - fu
            fr[16, sl] = (plsc.load_gather(t_cgd, [u0i, _spl_i(0)]) * gfu
                          + plsc.load_gather(t_cgd, [u1i, _spl_i(0)]) * fu)
            fr[17, sl] = (plsc.load_gather(t_cgd, [u0i, _spl_i(1)]) * gfu
                          + plsc.load_gather(t_cgd, [u1i, _spl_i(1)]) * fu)
            r0 = col(0, lanes, 0)
            sc0 = col(4, lanes, 0)
            cs, fu, fv = _bilin_idx(r0, sc0, 64)
            w00 = (1.0 - fu) * (1.0 - fv)
            w10 = fu * (1.0 - fv)
            w01 = (1.0 - fu) * fv
            w11 = fu * fv
            m0 = col(3, lanes, 0)
            lum0 = col(7, lanes, 0)
            cd = (1.0 - m0) * lum0
            for ch in range(3):
                t00 = plsc.load_gather(t_rs, [cs[0], _spl_i(ch)])
                t01 = plsc.load_gather(t_rs, [cs[1], _spl_i(ch)])
                t10 = plsc.load_gather(t_rs, [cs[2], _spl_i(ch)])
                t11 = plsc.load_gather(t_rs, [cs[3], _spl_i(ch)])
                rse = t00 * w00 + t10 * w10 + t01 * w01 + t11 * w11
                rse = jnp.clip(rse, 0.0, CLIP_HI)
                fr[8 + ch, sl] = rse * cd
            sp0 = col(5, lanes, 0)
            sp1 = col(5, lanes, 1)
            sh0 = col(6, lanes, 0)
            sh1 = col(6, lanes, 1)
            fr[11, sl] = sp0 * 0.1 * (1.0 - m0) * sp1 + m0 * lum0
            fr[12, sl] = sp0 * 0.1 * (1.0 - m0) * (1.0 - sp1)
            fr[13, sl] = sh0 * (1.0 - m0) * sh1
            fr[14, sl] = sh0 * (1.0 - m0) * (1.0 - sh1)
            fr[15, sl] = 0.0625 * col(7, lanes, 1)

        for cp in cps_ra:
            cp.wait()
        for i in range(VPB):
            sl = pl.ds(i * 16, 16)
            lanes = lax.iota(I32, 16) + i * 16
            ra0 = jnp.clip(g[pl.ds(R_RA * B + i * 16, 16)], 0.0, CLIP_HI)
            ra1 = jnp.clip(g[pl.ds((R_RA + 1) * B + i * 16, 16)], 0.0, CLIP_HI)
            cs, fu, fv = _bilin_idx(ra0, ra1, 1024)
            for c in range(4):
                e = cs[c] * 3
                for ch in range(3):
                    idxb[pl.ds((R_RAD + 3 * c + ch) * B + i * 16, 16)] = e + ch
            fr[4, sl] = fu
            fr[5, sl] = fv
            nd0 = col(2, lanes, 0)
            nd1 = col(2, lanes, 1)
            cs, fu, fv = _bilin_idx(nd0 * nd1, ra0, 2048)
            for c in range(4):
                idxb[pl.ds((R_LVR + c) * B + i * 16, 16)] = cs[c]
            fr[6, sl] = fu
            fr[7, sl] = fv
        cps_b.append(pltpu.async_copy(
            rad_h.at[idxb.at[pl.ds(R_RAD * B, 12 * B)]],
            g.at[pl.ds(R_RAD * B, 12 * B)], sem_b))
        cps_b.append(pltpu.async_copy(
            lvr_h.at[idxb.at[pl.ds(R_LVR * B, 4 * B)]],
            g.at[pl.ds(R_LVR * B, 4 * B)], sem_b))
        for cp in cps_b:
            cp.wait()

        for i in range(VPB):
            sl = pl.ds(i * 16, 16)
            lanes = lax.iota(I32, 16) + i * 16

            def bw(fu, fv):
                return ((1.0 - fu) * (1.0 - fv), fu * (1.0 - fv),
                        (1.0 - fu) * fv, fu * fv)

            w = bw(fr[0, sl], fr[1, sl])
            nlv = []
            for ch in range(4):
                acc = (g[pl.ds((R_LV + ch) * B + i * 16, 16)] * w[0]
                       + g[pl.ds((R_LV + 8 + ch) * B + i * 16, 16)] * w[1]
                       + g[pl.ds((R_LV + 4 + ch) * B + i * 16, 16)] * w[2]
                       + g[pl.ds((R_LV + 12 + ch) * B + i * 16, 16)] * w[3])
                nlv.append(jnp.clip(acc, 0.0, CLIP_HI))
            w = bw(fr[2, sl], fr[3, sl])
            nhl = []
            for ch in range(4):
                acc = (g[pl.ds((R_NHL + ch) * B + i * 16, 16)] * w[0]
                       + g[pl.ds((R_NHL + 8 + ch) * B + i * 16, 16)] * w[1]
                       + g[pl.ds((R_NHL + 4 + ch) * B + i * 16, 16)] * w[2]
                       + g[pl.ds((R_NHL + 12 + ch) * B + i * 16, 16)] * w[3])
                nhl.append(jnp.clip(acc, 0.0, CLIP_HI))
            w = bw(fr[4, sl], fr[5, sl])
            rac = []
            for ch in range(3):
                rac.append(g[pl.ds((R_RAD + ch) * B + i * 16, 16)] * w[0]
                           + g[pl.ds((R_RAD + 6 + ch) * B + i * 16, 16)] * w[1]
                           + g[pl.ds((R_RAD + 3 + ch) * B + i * 16, 16)] * w[2]
                           + g[pl.ds((R_RAD + 9 + ch) * B + i * 16, 16)] * w[3])
            w = bw(fr[6, sl], fr[7, sl])
            vterm = (g[pl.ds(R_LVR * B + i * 16, 16)] * w[0]
                     + g[pl.ds((R_LVR + 2) * B + i * 16, 16)] * w[1]
                     + g[pl.ds((R_LVR + 1) * B + i * 16, 16)] * w[2]
                     + g[pl.ds((R_LVR + 3) * B + i * 16, 16)] * w[3])
            vterm = jnp.clip(vterm, 0.0, CLIP_HI)

            a1 = col(1, lanes, 1)
            r1 = col(0, lanes, 1)
            nd1 = col(2, lanes, 1)
            m1 = col(3, lanes, 1)
            noh2 = r1 * r1
            den = rac[0] * (a1 * a1) + rac[1] * noh2 + rac[2]
            dterm = 0.25 / (nd1 * (den * den))
            cm0 = fr[11, sl]
            cm1 = fr[12, sl]
            x = (fr[8, sl] * m1 * nlv[1]
                 + fr[9, sl] * nlv[2]
                 + fr[10, sl] * nlv[3]
                 + cm0 * vterm * nhl[2] * dterm
                 + fr[13, sl] * nhl[0])
            y = (((1.0 - cm1) * nhl[3] + cm1) * vterm * dterm
                 + fr[14, sl] * nhl[0]
                 + fr[15, sl] * nhl[1] * nlv[0]
                 / (nd1 * (fr[16, sl] * noh2 + fr[17, sl])))
            ox[sl] = x
            oy[sl] = y

        pltpu.sync_copy(ox, outx_h.at[pl.ds(base, B)])
        pltpu.sync_copy(oy, outy_h.at[pl.ds(base, B)])
        return carry

    lax.fori_loop(0, NBLK, body, 0)


_sc_kernel = functools.partial(
    pl.kernel,
    out_type=(jax.ShapeDtypeStruct((N,), F32),
              jax.ShapeDtypeStruct((N,), F32)),
    mesh=plsc.VectorSubcoreMesh(core_axis_name="c", subcore_axis_name="s",
                                num_cores=2, num_subcores=16),
    compiler_params=pltpu.CompilerParams(needs_layout_passes=False,
                                         use_tc_tiling_on_sc=False),
    scratch_types=[
        pltpu.VMEM((8 * B, 2), F32),
        pltpu.VMEM((NSTREAM * B,), I32),
        pltpu.VMEM((NSTREAM * B,), F32),
        pltpu.VMEM((18, B), F32),
        pltpu.VMEM((4096, 3), F32),
        pltpu.VMEM((2048,), F32),
        pltpu.VMEM((2048, 2), F32),
        pltpu.VMEM((B,), F32),
        pltpu.VMEM((B,), F32),
        pltpu.SemaphoreType.DMA,
        pltpu.SemaphoreType.DMA,
        pltpu.SemaphoreType.DMA,
    ],
)(_sc_body)


def kernel(roughNoh, anisoToh, nDotLV, tDotLV, metalLoh, subCg, spst, shst,
           lumCs, tabRAEnc, tabCgEnc, tabRSEnc, tabRADec, tabCgDec, tabLVR,
           tabLV, tabNHL):
    del tDotLV
    def _flat(t, n):
        return jnp.maximum(t.reshape(n), 0.0)

    x, y = _sc_kernel(
        roughNoh, anisoToh, nDotLV, metalLoh, subCg, spst, shst, lumCs,
        _flat(tabRAEnc, 2048 * 2048 * 2),
        tabCgEnc.reshape(2048),
        tabRSEnc.reshape(64 * 64, 3),
        _flat(tabRADec, 1024 * 1024 * 3),
        tabCgDec,
        _flat(tabLVR, 2048 * 2048),
        _flat(tabLV, 1024 * 1024 * 4),
        _flat(tabNHL, 1024 * 1024 * 4),
    )
    return (x[:, None], y[:, None])


# speedup vs baseline: 1.0020x; 1.0020x over previous
import functools

import jax
import jax.numpy as jnp
from jax import lax
from jax.experimental import pallas as pl
from jax.experimental.pallas import tpu as pltpu
from jax.experimental.pallas import tpu_sc as plsc

N = 1048576
B = 256
VPB = B // 16
NW = 32
PW = N // NW
NBLK = PW // B

F32 = jnp.float32
I32 = jnp.int32

CLIP_HI = 1.0 - 1e-6

R_RA = 0
R_LV = 2
R_NHL = 18
R_RAD = 34
R_LVR = 46
NSTREAM = 50


def _spl_i(v):
    return jnp.full((16,), v, I32)


def _rnd_even(u):
    f = u.astype(I32)
    r = u - f.astype(F32)
    odd = (f & 1) == 1
    up = (r > 0.5) | ((r == 0.5) & odd)
    return jnp.where(up, f + 1, f)


def _bilin_idx(x0, x1, res):
    u = jnp.clip(x0, 0.0, 1.0) * (res - 1)
    v = jnp.clip(x1, 0.0, 1.0) * (res - 1)
    u0i = u.astype(I32)
    v0i = v.astype(I32)
    u1i = jnp.minimum(u0i + 1, res - 1)
    v1i = jnp.minimum(v0i + 1, res - 1)
    fu = u - u0i.astype(F32)
    fv = v - v0i.astype(F32)
    c00 = u0i * res + v0i
    c01 = u0i * res + v1i
    c10 = u1i * res + v0i
    c11 = u1i * res + v1i
    return (c00, c01, c10, c11), fu, fv


def _sc_body(rough_h, aniso_h, ndlv_h, metal_h, subcg_h, spst_h, shst_h,
             lum_h, raenc_h, cge_h, rs_h, rad_h, cgd_h, lvr_h, lv_h, nhl_h,
             outx_h, outy_h,
             inb, idxb, g, fr, t_rs, t_cge, t_cgd, ox, oy,
             sem_in, sem_ra, sem_b):
    wid = lax.axis_index("s") * 2 + lax.axis_index("c")

    pltpu.sync_copy(rs_h, t_rs)
    pltpu.sync_copy(cge_h, t_cge)
    pltpu.sync_copy(cgd_h, t_cgd)

    inputs_h = (rough_h, aniso_h, ndlv_h, metal_h, subcg_h, spst_h, shst_h,
                lum_h)

    def col(k, lanes, c):
        return plsc.load_gather(inb, [lanes + k * B, _spl_i(c)])

    def body(blk, carry):
        base = wid * PW + blk * B

        cps = [pltpu.async_copy(inp.at[pl.ds(base, B)],
                                inb.at[pl.ds(k * B, B)], sem_in)
               for k, inp in enumerate(inputs_h)]
        for cp in cps:
            cp.wait()

        for i in range(VPB):
            sl = pl.ds(i * 16, 16)
            lanes = lax.iota(I32, 16) + i * 16
            r0 = jnp.clip(col(0, lanes, 0), 0.0, 1.0) * 2047.0
            a0 = jnp.clip(col(1, lanes, 0), 0.0, 1.0) * 2047.0
            e = (_rnd_even(r0) * 2048 + _rnd_even(a0)) * 2
            idxb[pl.ds(R_RA * B + i * 16, 16)] = e
            idxb[pl.ds((R_RA + 1) * B + i * 16, 16)] = e + 1
        cps_ra = [pltpu.async_copy(
            raenc_h.at[idxb.at[pl.ds(R_RA * B, 2 * B)]],
            g.at[pl.ds(R_RA * B, 2 * B)], sem_ra)]

        for i in range(VPB):
            sl = pl.ds(i * 16, 16)
            lanes = lax.iota(I32, 16) + i * 16
            nd0 = col(2, lanes, 0)
            nd1 = col(2, lanes, 1)
            cs, fu, fv = _bilin_idx(nd0, nd1, 1024)
            for c in range(4):
                e = cs[c] * 4
                for ch in range(4):
                    idxb[pl.ds((R_LV + 4 * c + ch) * B + i * 16, 16)] = e + ch
            fr[0, sl] = fu
            fr[1, sl] = fv
            m1 = col(3, lanes, 1)
            cs, fu, fv = _bilin_idx(m1, nd0, 1024)
            for c in range(4):
                e = cs[c] * 4
                for ch in range(4):
                    idxb[pl.ds((R_NHL + 4 * c + ch) * B + i * 16, 16)] = e + ch
            fr[2, sl] = fu
            fr[3, sl] = fv
        cps_b = [pltpu.async_copy(
            lv_h.at[idxb.at[pl.ds(R_LV * B, 16 * B)]],
            g.at[pl.ds(R_LV * B, 16 * B)], sem_b)]
        cps_b.append(pltpu.async_copy(
            nhl_h.at[idxb.at[pl.ds(R_NHL * B, 16 * B)]],
            g.at[pl.ds(R_NHL * B, 16 * B)], sem_b))

        for i in range(VPB):
            sl = pl.ds(i * 16, 16)
            lanes = lax.iota(I32, 16) + i * 16
            sc1 = jnp.clip(col(4, lanes, 1), 0.0, 1.0) * 2047.0
            cgv = plsc.load_gather(t_cge, [_rnd_even(sc1)])
            cg0 = jnp.clip(cgv, 0.0, CLIP_HI)
            u = jnp.clip(cg0, 0.0, 1.0) * 2047.0
            u0i = u.astype(I32)
            u1i = jnp.minimum(u0i + 1, 2047)
            fu = u - u0i.astype(F32)
            gfu = 1.0 - fu
            fr[16, sl] = (plsc.load_gather(t_cgd, [u0i, _spl_i(0)]) * gfu
                          + plsc.load_gather(t_cgd, [u1i, _spl_i(0)]) * fu)
            fr[17, sl] = (plsc.load_gather(t_cgd, [u0i, _spl_i(1)]) * gfu
                          + plsc.load_gather(t_cgd, [u1i, _spl_i(1)]) * fu)
            r0 = col(0, lanes, 0)
            sc0 = col(4, lanes, 0)
            cs, fu, fv = _bilin_idx(r0, sc0, 64)
            w00 = (1.0 - fu) * (1.0 - fv)
            w10 = fu * (1.0 - fv)
            w01 = (1.0 - fu) * fv
            w11 = fu * fv
            m0 = col(3, lanes, 0)
            lum0 = col(7, lanes, 0)
            cd = (1.0 - m0) * lum0
            for ch in range(3):
                t00 = plsc.load_gather(t_rs, [cs[0], _spl_i(ch)])
                t01 = plsc.load_gather(t_rs, [cs[1], _spl_i(ch)])
                t10 = plsc.load_gather(t_rs, [cs[2], _spl_i(ch)])
                t11 = plsc.load_gather(t_rs, [cs[3], _spl_i(ch)])
                rse = t00 * w00 + t10 * w10 + t01 * w01 + t11 * w11
                rse = jnp.clip(rse, 0.0, CLIP_HI)
                fr[8 + ch, sl] = rse * cd
            sp0 = col(5, lanes, 0)
            sp1 = col(5, lanes, 1)
            sh0 = col(6, lanes, 0)
            sh1 = col(6, lanes, 1)
            fr[11, sl] = sp0 * 0.1 * (1.0 - m0) * sp1 + m0 * lum0
            fr[12, sl] = sp0 * 0.1 * (1.0 - m0) * (1.0 - sp1)
            fr[13, sl] = sh0 * (1.0 - m0) * sh1
            fr[14, sl] = sh0 * (1.0 - m0) * (1.0 - sh1)
            fr[15, sl] = 0.0625 * col(7, lanes, 1)

        for cp in cps_ra:
            cp.wait()
        for i in range(VPB):
            sl = pl.ds(i * 16, 16)
            lanes = lax.iota(I32, 16) + i * 16
            ra0 = jnp.clip(-g[pl.ds(R_RA * B + i * 16, 16)], 0.0, CLIP_HI)
            ra1 = jnp.clip(-g[pl.ds((R_RA + 1) * B + i * 16, 16)], 0.0, CLIP_HI)
            cs, fu, fv = _bilin_idx(ra0, ra1, 1024)
            for c in range(4):
                e = cs[c] * 3
                for ch in range(3):
                    idxb[pl.ds((R_RAD + 3 * c + ch) * B + i * 16, 16)] = e + ch
            fr[4, sl] = fu
            fr[5, sl] = fv
            nd0 = col(2, lanes, 0)
            nd1 = col(2, lanes, 1)
            cs, fu, fv = _bilin_idx(nd0 * nd1, ra0, 2048)
            for c in range(4):
                idxb[pl.ds((R_LVR + c) * B + i * 16, 16)] = cs[c]
            fr[6, sl] = fu
            fr[7, sl] = fv
        cps_b.append(pltpu.async_copy(
            rad_h.at[idxb.at[pl.ds(R_RAD * B, 12 * B)]],
            g.at[pl.ds(R_RAD * B, 12 * B)], sem_b))
        cps_b.append(pltpu.async_copy(
            lvr_h.at[idxb.at[pl.ds(R_LVR * B, 4 * B)]],
            g.at[pl.ds(R_LVR * B, 4 * B)], sem_b))
        for cp in cps_b:
            cp.wait()

        for i in range(VPB):
            sl = pl.ds(i * 16, 16)
            lanes = lax.iota(I32, 16) + i * 16

            def bw(fu, fv):
                return ((1.0 - fu) * (1.0 - fv), fu * (1.0 - fv),
                        (1.0 - fu) * fv, fu * fv)

            w = bw(fr[0, sl], fr[1, sl])
            nlv = []
            for ch in range(4):
                acc = (g[pl.ds((R_LV + ch) * B + i * 16, 16)] * w[0]
                       + g[pl.ds((R_LV + 8 + ch) * B + i * 16, 16)] * w[1]
                       + g[pl.ds((R_LV + 4 + ch) * B + i * 16, 16)] * w[2]
                       + g[pl.ds((R_LV + 12 + ch) * B + i * 16, 16)] * w[3])
                nlv.append(jnp.clip(-acc, 0.0, CLIP_HI))
            w = bw(fr[2, sl], fr[3, sl])
            nhl = []
            for ch in range(4):
                acc = (g[pl.ds((R_NHL + ch) * B + i * 16, 16)] * w[0]
                       + g[pl.ds((R_NHL + 8 + ch) * B + i * 16, 16)] * w[1]
                       + g[pl.ds((R_NHL + 4 + ch) * B + i * 16, 16)] * w[2]
                       + g[pl.ds((R_NHL + 12 + ch) * B + i * 16, 16)] * w[3])
                nhl.append(jnp.clip(-acc, 0.0, CLIP_HI))
            w = bw(fr[4, sl], fr[5, sl])
            rac = []
            for ch in range(3):
                rac.append(-(g[pl.ds((R_RAD + ch) * B + i * 16, 16)] * w[0]
                           + g[pl.ds((R_RAD + 6 + ch) * B + i * 16, 16)] * w[1]
                           + g[pl.ds((R_RAD + 3 + ch) * B + i * 16, 16)] * w[2]
                           + g[pl.ds((R_RAD + 9 + ch) * B + i * 16, 16)] * w[3]))
            w = bw(fr[6, sl], fr[7, sl])
            vterm = (g[pl.ds(R_LVR * B + i * 16, 16)] * w[0]
                     + g[pl.ds((R_LVR + 2) * B + i * 16, 16)] * w[1]
                     + g[pl.ds((R_LVR + 1) * B + i * 16, 16)] * w[2]
                     + g[pl.ds((R_LVR + 3) * B + i * 16, 16)] * w[3])
            vterm = jnp.clip(-vterm, 0.0, CLIP_HI)

            a1 = col(1, lanes, 1)
            r1 = col(0, lanes, 1)
            nd1 = col(2, lanes, 1)
            m1 = col(3, lanes, 1)
            noh2 = r1 * r1
            den = rac[0] * (a1 * a1) + rac[1] * noh2 + rac[2]
            dterm = 0.25 / (nd1 * (den * den))
            cm0 = fr[11, sl]
            cm1 = fr[12, sl]
            x = (fr[8, sl] * m1 * nlv[1]
                 + fr[9, sl] * nlv[2]
                 + fr[10, sl] * nlv[3]
                 + cm0 * vterm * nhl[2] * dterm
                 + fr[13, sl] * nhl[0])
            y = (((1.0 - cm1) * nhl[3] + cm1) * vterm * dterm
                 + fr[14, sl] * nhl[0]
                 + fr[15, sl] * nhl[1] * nlv[0]
                 / (nd1 * (fr[16, sl] * noh2 + fr[17, sl])))
            ox[sl] = x
            oy[sl] = y

        pltpu.sync_copy(ox, outx_h.at[pl.ds(base, B)])
        pltpu.sync_copy(oy, outy_h.at[pl.ds(base, B)])
        return carry

    lax.fori_loop(0, NBLK, body, 0)


_sc_kernel = functools.partial(
    pl.kernel,
    out_type=(jax.ShapeDtypeStruct((N,), F32),
              jax.ShapeDtypeStruct((N,), F32)),
    mesh=plsc.VectorSubcoreMesh(core_axis_name="c", subcore_axis_name="s",
                                num_cores=2, num_subcores=16),
    compiler_params=pltpu.CompilerParams(needs_layout_passes=False,
                                         use_tc_tiling_on_sc=False),
    scratch_types=[
        pltpu.VMEM((8 * B, 2), F32),
        pltpu.VMEM((NSTREAM * B,), I32),
        pltpu.VMEM((NSTREAM * B,), F32),
        pltpu.VMEM((18, B), F32),
        pltpu.VMEM((4096, 3), F32),
        pltpu.VMEM((2048,), F32),
        pltpu.VMEM((2048, 2), F32),
        pltpu.VMEM((B,), F32),
        pltpu.VMEM((B,), F32),
        pltpu.SemaphoreType.DMA,
        pltpu.SemaphoreType.DMA,
        pltpu.SemaphoreType.DMA,
    ],
)(_sc_body)


def kernel(roughNoh, anisoToh, nDotLV, tDotLV, metalLoh, subCg, spst, shst,
           lumCs, tabRAEnc, tabCgEnc, tabRSEnc, tabRADec, tabCgDec, tabLVR,
           tabLV, tabNHL):
    del tDotLV
    def _flat(t, n):
        return 0.0 - t.reshape(n)

    x, y = _sc_kernel(
        roughNoh, anisoToh, nDotLV, metalLoh, subCg, spst, shst, lumCs,
        _flat(tabRAEnc, 2048 * 2048 * 2),
        tabCgEnc.reshape(2048),
        tabRSEnc.reshape(64 * 64, 3),
        _flat(tabRADec, 1024 * 1024 * 3),
        tabCgDec,
        _flat(tabLVR, 2048 * 2048),
        _flat(tabLV, 1024 * 1024 * 4),
        _flat(tabNHL, 1024 * 1024 * 4),
    )
    return (x[:, None], y[:, None])


# speedup vs baseline: 1.1071x; 1.1048x over previous
import functools

import jax
import jax.numpy as jnp
from jax import lax
from jax.experimental import pallas as pl
from jax.experimental.pallas import tpu as pltpu
from jax.experimental.pallas import tpu_sc as plsc

N = 1048576
B = 256
VPB = B // 16
NW = 32
PW = N // NW
NBLK = PW // B

F32 = jnp.float32
I32 = jnp.int32

CLIP_HI = 1.0 - 1e-6

R_RA = 0
R_LV = 2
R_NHL = 18
R_RAD = 34
R_LVR = 46
NSTREAM = 50


def _spl_i(v):
    return jnp.full((16,), v, I32)


def _rnd_even(u):
    f = u.astype(I32)
    r = u - f.astype(F32)
    odd = (f & 1) == 1
    up = (r > 0.5) | ((r == 0.5) & odd)
    return jnp.where(up, f + 1, f)


def _bilin_idx(x0, x1, res):
    u = jnp.clip(x0, 0.0, 1.0) * (res - 1)
    v = jnp.clip(x1, 0.0, 1.0) * (res - 1)
    u0i = u.astype(I32)
    v0i = v.astype(I32)
    u1i = jnp.minimum(u0i + 1, res - 1)
    v1i = jnp.minimum(v0i + 1, res - 1)
    fu = u - u0i.astype(F32)
    fv = v - v0i.astype(F32)
    c00 = u0i * res + v0i
    c01 = u0i * res + v1i
    c10 = u1i * res + v0i
    c11 = u1i * res + v1i
    return (c00, c01, c10, c11), fu, fv


def _sc_body(rough_h, aniso_h, ndlv_h, metal_h, subcg_h, spst_h, shst_h,
             lum_h, raenc_h, cge_h, rs_h, rad_h, cgd_h, lvr_h, lv_h, nhl_h,
             outx_h, outy_h,
             inb, idxb, g, fr, t_rs, t_cge, t_cgd, ox, oy,
             sem_in, sem_ra, sem_b):
    wid = lax.axis_index("s") * 2 + lax.axis_index("c")

    pltpu.sync_copy(rs_h, t_rs)
    pltpu.sync_copy(cge_h, t_cge)
    pltpu.sync_copy(cgd_h, t_cgd)

    inputs_h = (rough_h, aniso_h, ndlv_h, metal_h, subcg_h, spst_h, shst_h,
                lum_h)

    def col(k, lanes, c):
        return -plsc.load_gather(inb, [lanes * 2 + (k * 2 * B + c)])

    def body(blk, carry):
        base = wid * PW + blk * B

        cps = [pltpu.async_copy(inp.at[pl.ds(base * 2, 2 * B)],
                                inb.at[pl.ds(k * 2 * B, 2 * B)], sem_in)
               for k, inp in enumerate(inputs_h)]
        for cp in cps:
            cp.wait()

        for i in range(VPB):
            sl = pl.ds(i * 16, 16)
            lanes = lax.iota(I32, 16) + i * 16
            r0 = jnp.clip(col(0, lanes, 0), 0.0, 1.0) * 2047.0
            a0 = jnp.clip(col(1, lanes, 0), 0.0, 1.0) * 2047.0
            e = (_rnd_even(r0) * 2048 + _rnd_even(a0)) * 2
            idxb[pl.ds(R_RA * B + i * 16, 16)] = e
            idxb[pl.ds((R_RA + 1) * B + i * 16, 16)] = e + 1
        cps_ra = [pltpu.async_copy(
            raenc_h.at[idxb.at[pl.ds(R_RA * B, 2 * B)]],
            g.at[pl.ds(R_RA * B, 2 * B)], sem_ra)]

        for i in range(VPB):
            sl = pl.ds(i * 16, 16)
            lanes = lax.iota(I32, 16) + i * 16
            nd0 = col(2, lanes, 0)
            nd1 = col(2, lanes, 1)
            cs, fu, fv = _bilin_idx(nd0, nd1, 1024)
            for c in range(4):
                e = cs[c] * 4
                for ch in range(4):
                    idxb[pl.ds((R_LV + 4 * c + ch) * B + i * 16, 16)] = e + ch
            fr[0, sl] = fu
            fr[1, sl] = fv
            m1 = col(3, lanes, 1)
            cs, fu, fv = _bilin_idx(m1, nd0, 1024)
            for c in range(4):
                e = cs[c] * 4
                for ch in range(4):
                    idxb[pl.ds((R_NHL + 4 * c + ch) * B + i * 16, 16)] = e + ch
            fr[2, sl] = fu
            fr[3, sl] = fv
        cps_b = [pltpu.async_copy(
            lv_h.at[idxb.at[pl.ds(R_LV * B, 16 * B)]],
            g.at[pl.ds(R_LV * B, 16 * B)], sem_b)]
        cps_b.append(pltpu.async_copy(
            nhl_h.at[idxb.at[pl.ds(R_NHL * B, 16 * B)]],
            g.at[pl.ds(R_NHL * B, 16 * B)], sem_b))

        for i in range(VPB):
            sl = pl.ds(i * 16, 16)
            lanes = lax.iota(I32, 16) + i * 16
            sc1 = jnp.clip(col(4, lanes, 1), 0.0, 1.0) * 2047.0
            cgv = plsc.load_gather(t_cge, [_rnd_even(sc1)])
            cg0 = jnp.clip(cgv, 0.0, CLIP_HI)
            u = jnp.clip(cg0, 0.0, 1.0) * 2047.0
            u0i = u.astype(I32)
            u1i = jnp.minimum(u0i + 1, 2047)
            fu = u - u0i.astype(F32)
            gfu = 1.0 - fu
            fr[16, sl] = (plsc.load_gather(t_cgd, [u0i, _spl_i(0)]) * gfu
                          + plsc.load_gather(t_cgd, [u1i, _spl_i(0)]) * fu)
            fr[17, sl] = (plsc.load_gather(t_cgd, [u0i, _spl_i(1)]) * gfu
                          + plsc.load_gather(t_cgd, [u1i, _spl_i(1)]) * fu)
            r0 = col(0, lanes, 0)
            sc0 = col(4, lanes, 0)
            cs, fu, fv = _bilin_idx(r0, sc0, 64)
            w00 = (1.0 - fu) * (1.0 - fv)
            w10 = fu * (1.0 - fv)
            w01 = (1.0 - fu) * fv
            w11 = fu * fv
            m0 = col(3, lanes, 0)
            lum0 = col(7, lanes, 0)
            cd = (1.0 - m0) * lum0
            for ch in range(3):
                t00 = plsc.load_gather(t_rs, [cs[0], _spl_i(ch)])
                t01 = plsc.load_gather(t_rs, [cs[1], _spl_i(ch)])
                t10 = plsc.load_gather(t_rs, [cs[2], _spl_i(ch)])
                t11 = plsc.load_gather(t_rs, [cs[3], _spl_i(ch)])
                rse = t00 * w00 + t10 * w10 + t01 * w01 + t11 * w11
                rse = jnp.clip(rse, 0.0, CLIP_HI)
                fr[8 + ch, sl] = rse * cd
            sp0 = col(5, lanes, 0)
            sp1 = col(5, lanes, 1)
            sh0 = col(6, lanes, 0)
            sh1 = col(6, lanes, 1)
            fr[11, sl] = sp0 * 0.1 * (1.0 - m0) * sp1 + m0 * lum0
            fr[12, sl] = sp0 * 0.1 * (1.0 - m0) * (1.0 - sp1)
            fr[13, sl] = sh0 * (1.0 - m0) * sh1
            fr[14, sl] = sh0 * (1.0 - m0) * (1.0 - sh1)
            fr[15, sl] = 0.0625 * col(7, lanes, 1)

        for cp in cps_ra:
            cp.wait()
        for i in range(VPB):
            sl = pl.ds(i * 16, 16)
            lanes = lax.iota(I32, 16) + i * 16
            ra0 = jnp.clip(-g[pl.ds(R_RA * B + i * 16, 16)], 0.0, CLIP_HI)
            ra1 = jnp.clip(-g[pl.ds((R_RA + 1) * B + i * 16, 16)], 0.0, CLIP_HI)
            cs, fu, fv = _bilin_idx(ra0, ra1, 1024)
            for c in range(4):
                e = cs[c] * 3
                for ch in range(3):
                    idxb[pl.ds((R_RAD + 3 * c + ch) * B + i * 16, 16)] = e + ch
            fr[4, sl] = fu
            fr[5, sl] = fv
            nd0 = col(2, lanes, 0)
            nd1 = col(2, lanes, 1)
            cs, fu, fv = _bilin_idx(nd0 * nd1, ra0, 2048)
            for c in range(4):
                idxb[pl.ds((R_LVR + c) * B + i * 16, 16)] = cs[c]
            fr[6, sl] = fu
            fr[7, sl] = fv
        cps_b.append(pltpu.async_copy(
            rad_h.at[idxb.at[pl.ds(R_RAD * B, 12 * B)]],
            g.at[pl.ds(R_RAD * B, 12 * B)], sem_b))
        cps_b.append(pltpu.async_copy(
            lvr_h.at[idxb.at[pl.ds(R_LVR * B, 4 * B)]],
            g.at[pl.ds(R_LVR * B, 4 * B)], sem_b))
        for cp in cps_b:
            cp.wait()

        for i in range(VPB):
            sl = pl.ds(i * 16, 16)
            lanes = lax.iota(I32, 16) + i * 16

            def bw(fu, fv):
                return ((1.0 - fu) * (1.0 - fv), fu * (1.0 - fv),
                        (1.0 - fu) * fv, fu * fv)

            w = bw(fr[0, sl], fr[1, sl])
            nlv = []
            for ch in range(4):
                acc = (g[pl.ds((R_LV + ch) * B + i * 16, 16)] * w[0]
                       + g[pl.ds((R_LV + 8 + ch) * B + i * 16, 16)] * w[1]
                       + g[pl.ds((R_LV + 4 + ch) * B + i * 16, 16)] * w[2]
                       + g[pl.ds((R_LV + 12 + ch) * B + i * 16, 16)] * w[3])
                nlv.append(jnp.clip(-acc, 0.0, CLIP_HI))
            w = bw(fr[2, sl], fr[3, sl])
            nhl = []
            for ch in range(4):
                acc = (g[pl.ds((R_NHL + ch) * B + i * 16, 16)] * w[0]
                       + g[pl.ds((R_NHL + 8 + ch) * B + i * 16, 16)] * w[1]
                       + g[pl.ds((R_NHL + 4 + ch) * B + i * 16, 16)] * w[2]
                       + g[pl.ds((R_NHL + 12 + ch) * B + i * 16, 16)] * w[3])
                nhl.append(jnp.clip(-acc, 0.0, CLIP_HI))
            w = bw(fr[4, sl], fr[5, sl])
            rac = []
            for ch in range(3):
                rac.append(-(g[pl.ds((R_RAD + ch) * B + i * 16, 16)] * w[0]
                           + g[pl.ds((R_RAD + 6 + ch) * B + i * 16, 16)] * w[1]
                           + g[pl.ds((R_RAD + 3 + ch) * B + i * 16, 16)] * w[2]
                           + g[pl.ds((R_RAD + 9 + ch) * B + i * 16, 16)] * w[3]))
            w = bw(fr[6, sl], fr[7, sl])
            vterm = (g[pl.ds(R_LVR * B + i * 16, 16)] * w[0]
                     + g[pl.ds((R_LVR + 2) * B + i * 16, 16)] * w[1]
                     + g[pl.ds((R_LVR + 1) * B + i * 16, 16)] * w[2]
                     + g[pl.ds((R_LVR + 3) * B + i * 16, 16)] * w[3])
            vterm = jnp.clip(-vterm, 0.0, CLIP_HI)

            a1 = col(1, lanes, 1)
            r1 = col(0, lanes, 1)
            nd1 = col(2, lanes, 1)
            m1 = col(3, lanes, 1)
            noh2 = r1 * r1
            den = rac[0] * (a1 * a1) + rac[1] * noh2 + rac[2]
            dterm = 0.25 / (nd1 * (den * den))
            cm0 = fr[11, sl]
            cm1 = fr[12, sl]
            x = (fr[8, sl] * m1 * nlv[1]
                 + fr[9, sl] * nlv[2]
                 + fr[10, sl] * nlv[3]
                 + cm0 * vterm * nhl[2] * dterm
                 + fr[13, sl] * nhl[0])
            y = (((1.0 - cm1) * nhl[3] + cm1) * vterm * dterm
                 + fr[14, sl] * nhl[0]
                 + fr[15, sl] * nhl[1] * nlv[0]
                 / (nd1 * (fr[16, sl] * noh2 + fr[17, sl])))
            ox[sl] = x
            oy[sl] = y

        pltpu.sync_copy(ox, outx_h.at[pl.ds(base, B)])
        pltpu.sync_copy(oy, outy_h.at[pl.ds(base, B)])
        return carry

    lax.fori_loop(0, NBLK, body, 0)


_sc_kernel = functools.partial(
    pl.kernel,
    out_type=(jax.ShapeDtypeStruct((N,), F32),
              jax.ShapeDtypeStruct((N,), F32)),
    mesh=plsc.VectorSubcoreMesh(core_axis_name="c", subcore_axis_name="s",
                                num_cores=2, num_subcores=16),
    compiler_params=pltpu.CompilerParams(needs_layout_passes=False,
                                         use_tc_tiling_on_sc=False),
    scratch_types=[
        pltpu.VMEM((16 * B,), F32),
        pltpu.VMEM((NSTREAM * B,), I32),
        pltpu.VMEM((NSTREAM * B,), F32),
        pltpu.VMEM((18, B), F32),
        pltpu.VMEM((4096, 3), F32),
        pltpu.VMEM((2048,), F32),
        pltpu.VMEM((2048, 2), F32),
        pltpu.VMEM((B,), F32),
        pltpu.VMEM((B,), F32),
        pltpu.SemaphoreType.DMA,
        pltpu.SemaphoreType.DMA,
        pltpu.SemaphoreType.DMA,
    ],
)(_sc_body)


def kernel(roughNoh, anisoToh, nDotLV, tDotLV, metalLoh, subCg, spst, shst,
           lumCs, tabRAEnc, tabCgEnc, tabRSEnc, tabRADec, tabCgDec, tabLVR,
           tabLV, tabNHL):
    del tDotLV
    def _flat(t, n):
        return jax.lax.optimization_barrier((0.0 - t).reshape(n))

    x, y = _sc_kernel(
        _flat(roughNoh, 2 * N), _flat(anisoToh, 2 * N), _flat(nDotLV, 2 * N),
        _flat(metalLoh, 2 * N), _flat(subCg, 2 * N), _flat(spst, 2 * N),
        _flat(shst, 2 * N), _flat(lumCs, 2 * N),
        _flat(tabRAEnc, 2048 * 2048 * 2),
        tabCgEnc.reshape(2048),
        tabRSEnc.reshape(64 * 64, 3),
        _flat(tabRADec, 1024 * 1024 * 3),
        tabCgDec,
        _flat(tabLVR, 2048 * 2048),
        _flat(tabLV, 1024 * 1024 * 4),
        _flat(tabNHL, 1024 * 1024 * 4),
    )
    return (x[:, None], y[:, None])


# speedup vs baseline: 9.2266x; 8.3342x over previous
import functools

import jax
import jax.numpy as jnp
from jax import lax
from jax.experimental import pallas as pl
from jax.experimental.pallas import tpu as pltpu
from jax.experimental.pallas import tpu_sc as plsc

N = 1048576
B = 256
VPB = B // 16
NW = 32
PW = N // NW
NBLK = PW // B
NBI = B // 128

F32 = jnp.float32
I32 = jnp.int32

CLIP_HI = 1.0 - 1e-6

R_RA = 0
R_LV = 2
R_NHL = 18
R_RAD = 34
R_LVR = 46
NSTREAM = 50


def _spl_i(v):
    return jnp.full((16,), v, I32)


def _rnd_even(u):
    f = u.astype(I32)
    r = u - f.astype(F32)
    odd = (f & 1) == 1
    up = (r > 0.5) | ((r == 0.5) & odd)
    return jnp.where(up, f + 1, f)


def _bilin_uv(x0, x1, res):
    u = jnp.clip(x0, 0.0, 1.0) * (res - 1)
    v = jnp.clip(x1, 0.0, 1.0) * (res - 1)
    u0i = u.astype(I32)
    v0i = v.astype(I32)
    u1i = jnp.minimum(u0i + 1, res - 1)
    v1i = jnp.minimum(v0i + 1, res - 1)
    fu = u - u0i.astype(F32)
    fv = v - v0i.astype(F32)
    return u0i, u1i, v0i, v1i, fu, fv


def _sc_body(rough_h, aniso_h, ndlv_h, metal_h, subcg_h, spst_h, shst_h,
             lum_h, raenc_h, cge_h, rs_h, rad_h, cgd_h, lvr_h, lv_h, nhl_h,
             outx_h, outy_h,
             inb, idxb, g, fr, t_rs, t_cge, t_cgd, ox, oy,
             sem_in, sem_ra, sem_b):
    wid = lax.axis_index("s") * 2 + lax.axis_index("c")

    pltpu.sync_copy(rs_h, t_rs)
    pltpu.sync_copy(cge_h, t_cge)
    pltpu.sync_copy(cgd_h, t_cgd)

    inputs_h = (rough_h, aniso_h, ndlv_h, metal_h, subcg_h, spst_h, shst_h,
                lum_h)

    def col(k, i, c):
        blk = (i * 16) // 128
        w = lax.iota(I32, 16) + (i * 16) % 128
        return plsc.load_gather(inb, [_spl_i(k * NBI + blk), _spl_i(c), w])

    def body(blk_i, carry):
        base = wid * PW + blk_i * B

        cps = [pltpu.async_copy(inp.at[pl.ds(base // 128, NBI)],
                                inb.at[pl.ds(k * NBI, NBI)], sem_in)
               for k, inp in enumerate(inputs_h)]
        for cp in cps:
            cp.wait()

        for i in range(VPB):
            sl = pl.ds(i * 16, 16)
            r0 = jnp.clip(col(0, i, 0), 0.0, 1.0) * 2047.0
            a0 = jnp.clip(col(1, i, 0), 0.0, 1.0) * 2047.0
            ui = _rnd_even(r0)
            vi = _rnd_even(a0)
            e = ui * 4096 + (vi >> 7) * 256 + (vi & 127)
            idxb[pl.ds(R_RA * B + i * 16, 16)] = e
            idxb[pl.ds((R_RA + 1) * B + i * 16, 16)] = e + 128
        cps_ra = [pltpu.async_copy(
            raenc_h.at[idxb.at[pl.ds(R_RA * B, 2 * B)]],
            g.at[pl.ds(R_RA * B, 2 * B)], sem_ra)]

        def eb_lvnhl(u, v):
            return u * 4096 + (v >> 7) * 512 + (v & 127)

        for i in range(VPB):
            sl = pl.ds(i * 16, 16)
            nd0 = col(2, i, 0)
            nd1 = col(2, i, 1)
            u0, u1, v0, v1, fu, fv = _bilin_uv(nd0, nd1, 1024)
            for c, (uu, vv) in enumerate(((u0, v0), (u0, v1),
                                          (u1, v0), (u1, v1))):
                e = eb_lvnhl(uu, vv)
                for ch in range(4):
                    idxb[pl.ds((R_LV + 4 * c + ch) * B + i * 16, 16)] = (
                        e + ch * 128)
            fr[0, sl] = fu
            fr[1, sl] = fv
            m1 = col(3, i, 1)
            u0, u1, v0, v1, fu, fv = _bilin_uv(m1, nd0, 1024)
            for c, (uu, vv) in enumerate(((u0, v0), (u0, v1),
                                          (u1, v0), (u1, v1))):
                e = eb_lvnhl(uu, vv)
                for ch in range(4):
                    idxb[pl.ds((R_NHL + 4 * c + ch) * B + i * 16, 16)] = (
                        e + ch * 128)
            fr[2, sl] = fu
            fr[3, sl] = fv
        cps_b = [pltpu.async_copy(
            lv_h.at[idxb.at[pl.ds(R_LV * B, 16 * B)]],
            g.at[pl.ds(R_LV * B, 16 * B)], sem_b)]
        cps_b.append(pltpu.async_copy(
            nhl_h.at[idxb.at[pl.ds(R_NHL * B, 16 * B)]],
            g.at[pl.ds(R_NHL * B, 16 * B)], sem_b))

        for i in range(VPB):
            sl = pl.ds(i * 16, 16)
            sc1 = jnp.clip(col(4, i, 1), 0.0, 1.0) * 2047.0
            cgv = plsc.load_gather(t_cge, [_rnd_even(sc1)])
            cg0 = jnp.clip(cgv, 0.0, CLIP_HI)
            u = jnp.clip(cg0, 0.0, 1.0) * 2047.0
            u0i = u.astype(I32)
            u1i = jnp.minimum(u0i + 1, 2047)
            fu = u - u0i.astype(F32)
            gfu = 1.0 - fu
            fr[16, sl] = (plsc.load_gather(t_cgd, [u0i, _spl_i(0)]) * gfu
                          + plsc.load_gather(t_cgd, [u1i, _spl_i(0)]) * fu)
            fr[17, sl] = (plsc.load_gather(t_cgd, [u0i, _spl_i(1)]) * gfu
                          + plsc.load_gather(t_cgd, [u1i, _spl_i(1)]) * fu)
            r0 = col(0, i, 0)
            sc0 = col(4, i, 0)
            u0, u1, v0, v1, fu, fv = _bilin_uv(r0, sc0, 64)
            cs = (u0 * 64 + v0, u0 * 64 + v1, u1 * 64 + v0, u1 * 64 + v1)
            w00 = (1.0 - fu) * (1.0 - fv)
            w10 = fu * (1.0 - fv)
            w01 = (1.0 - fu) * fv
            w11 = fu * fv
            m0 = col(3, i, 0)
            lum0 = col(7, i, 0)
            cd = (1.0 - m0) * lum0
            for ch in range(3):
                t00 = plsc.load_gather(t_rs, [cs[0], _spl_i(ch)])
                t01 = plsc.load_gather(t_rs, [cs[1], _spl_i(ch)])
                t10 = plsc.load_gather(t_rs, [cs[2], _spl_i(ch)])
                t11 = plsc.load_gather(t_rs, [cs[3], _spl_i(ch)])
                rse = t00 * w00 + t10 * w10 + t01 * w01 + t11 * w11
                rse = jnp.clip(rse, 0.0, CLIP_HI)
                fr[8 + ch, sl] = rse * cd
            sp0 = col(5, i, 0)
            sp1 = col(5, i, 1)
            sh0 = col(6, i, 0)
            sh1 = col(6, i, 1)
            fr[11, sl] = sp0 * 0.1 * (1.0 - m0) * sp1 + m0 * lum0
            fr[12, sl] = sp0 * 0.1 * (1.0 - m0) * (1.0 - sp1)
            fr[13, sl] = sh0 * (1.0 - m0) * sh1
            fr[14, sl] = sh0 * (1.0 - m0) * (1.0 - sh1)
            fr[15, sl] = 0.0625 * col(7, i, 1)

        for cp in cps_ra:
            cp.wait()
        for i in range(VPB):
            sl = pl.ds(i * 16, 16)
            ra0 = jnp.clip(g[pl.ds(R_RA * B + i * 16, 16)], 0.0, CLIP_HI)
            ra1 = jnp.clip(g[pl.ds((R_RA + 1) * B + i * 16, 16)], 0.0,
                           CLIP_HI)
            u0, u1, v0, v1, fu, fv = _bilin_uv(ra0, ra1, 1024)
            for c, (uu, vv) in enumerate(((u0, v0), (u0, v1),
                                          (u1, v0), (u1, v1))):
                e = ((uu >> 3) * 8192 + (vv >> 7) * 1024
                     + (uu & 7) * 128 + (vv & 127))
                for ch in range(3):
                    idxb[pl.ds((R_RAD + 3 * c + ch) * B + i * 16, 16)] = (
                        e + ch * 1048576)
            fr[4, sl] = fu
            fr[5, sl] = fv
            nd0 = col(2, i, 0)
            nd1 = col(2, i, 1)
            u0, u1, v0, v1, fu, fv = _bilin_uv(nd0 * nd1, ra0, 2048)
            for c, (uu, vv) in enumerate(((u0, v0), (u0, v1),
                                          (u1, v0), (u1, v1))):
                idxb[pl.ds((R_LVR + c) * B + i * 16, 16)] = uu * 2048 + vv
            fr[6, sl] = fu
            fr[7, sl] = fv
        cps_b.append(pltpu.async_copy(
            rad_h.at[idxb.at[pl.ds(R_RAD * B, 12 * B)]],
            g.at[pl.ds(R_RAD * B, 12 * B)], sem_b))
        cps_b.append(pltpu.async_copy(
            lvr_h.at[idxb.at[pl.ds(R_LVR * B, 4 * B)]],
            g.at[pl.ds(R_LVR * B, 4 * B)], sem_b))
        for cp in cps_b:
            cp.wait()

        for i in range(VPB):
            sl = pl.ds(i * 16, 16)

            def bw(fu, fv):
                return ((1.0 - fu) * (1.0 - fv), fu * (1.0 - fv),
                        (1.0 - fu) * fv, fu * fv)

            w = bw(fr[0, sl], fr[1, sl])
            nlv = []
            for ch in range(4):
                acc = (g[pl.ds((R_LV + ch) * B + i * 16, 16)] * w[0]
                       + g[pl.ds((R_LV + 8 + ch) * B + i * 16, 16)] * w[1]
                       + g[pl.ds((R_LV + 4 + ch) * B + i * 16, 16)] * w[2]
                       + g[pl.ds((R_LV + 12 + ch) * B + i * 16, 16)] * w[3])
                nlv.append(jnp.clip(acc, 0.0, CLIP_HI))
            w = bw(fr[2, sl], fr[3, sl])
            nhl = []
            for ch in range(4):
                acc = (g[pl.ds((R_NHL + ch) * B + i * 16, 16)] * w[0]
                       + g[pl.ds((R_NHL + 8 + ch) * B + i * 16, 16)] * w[1]
                       + g[pl.ds((R_NHL + 4 + ch) * B + i * 16, 16)] * w[2]
                       + g[pl.ds((R_NHL + 12 + ch) * B + i * 16, 16)] * w[3])
                nhl.append(jnp.clip(acc, 0.0, CLIP_HI))
            w = bw(fr[4, sl], fr[5, sl])
            rac = []
            for ch in range(3):
                rac.append(g[pl.ds((R_RAD + ch) * B + i * 16, 16)] * w[0]
                           + g[pl.ds((R_RAD + 6 + ch) * B + i * 16, 16)] * w[1]
                           + g[pl.ds((R_RAD + 3 + ch) * B + i * 16, 16)] * w[2]
                           + g[pl.ds((R_RAD + 9 + ch) * B + i * 16, 16)] * w[3])
            w = bw(fr[6, sl], fr[7, sl])
            vterm = (g[pl.ds(R_LVR * B + i * 16, 16)] * w[0]
                     + g[pl.ds((R_LVR + 2) * B + i * 16, 16)] * w[1]
                     + g[pl.ds((R_LVR + 1) * B + i * 16, 16)] * w[2]
                     + g[pl.ds((R_LVR + 3) * B + i * 16, 16)] * w[3])
            vterm = jnp.clip(vterm, 0.0, CLIP_HI)

            a1 = col(1, i, 1)
            r1 = col(0, i, 1)
            nd1 = col(2, i, 1)
            m1 = col(3, i, 1)
            noh2 = r1 * r1
            den = rac[0] * (a1 * a1) + rac[1] * noh2 + rac[2]
            dterm = 0.25 / (nd1 * (den * den))
            cm0 = fr[11, sl]
            cm1 = fr[12, sl]
            x = (fr[8, sl] * m1 * nlv[1]
                 + fr[9, sl] * nlv[2]
                 + fr[10, sl] * nlv[3]
                 + cm0 * vterm * nhl[2] * dterm
                 + fr[13, sl] * nhl[0])
            y = (((1.0 - cm1) * nhl[3] + cm1) * vterm * dterm
                 + fr[14, sl] * nhl[0]
                 + fr[15, sl] * nhl[1] * nlv[0]
                 / (nd1 * (fr[16, sl] * noh2 + fr[17, sl])))
            ox[sl] = x
            oy[sl] = y

        pltpu.sync_copy(ox, outx_h.at[pl.ds(base, B)])
        pltpu.sync_copy(oy, outy_h.at[pl.ds(base, B)])
        return carry

    lax.fori_loop(0, NBLK, body, 0)


_sc_kernel = functools.partial(
    pl.kernel,
    out_type=(jax.ShapeDtypeStruct((N,), F32),
              jax.ShapeDtypeStruct((N,), F32)),
    mesh=plsc.VectorSubcoreMesh(core_axis_name="c", subcore_axis_name="s",
                                num_cores=2, num_subcores=16),
    compiler_params=pltpu.CompilerParams(needs_layout_passes=False,
                                         use_tc_tiling_on_sc=False),
    scratch_types=[
        pltpu.VMEM((8 * B // 128, 2, 128), F32),
        pltpu.VMEM((NSTREAM * B,), I32),
        pltpu.VMEM((NSTREAM * B,), F32),
        pltpu.VMEM((18, B), F32),
        pltpu.VMEM((4096, 3), F32),
        pltpu.VMEM((2048,), F32),
        pltpu.VMEM((2048, 2), F32),
        pltpu.VMEM((B,), F32),
        pltpu.VMEM((B,), F32),
        pltpu.SemaphoreType.DMA,
        pltpu.SemaphoreType.DMA,
        pltpu.SemaphoreType.DMA,
    ],
)(_sc_body)


def kernel(roughNoh, anisoToh, nDotLV, tDotLV, metalLoh, subCg, spst, shst,
           lumCs, tabRAEnc, tabCgEnc, tabRSEnc, tabRADec, tabCgDec, tabLVR,
           tabLV, tabNHL):
    del tDotLV

    def nv_in(t):
        return t.reshape(N // 128, 128, 2).transpose(0, 2, 1)

    raenc = (tabRAEnc.reshape(2048, 16, 128, 2).transpose(0, 1, 3, 2)
             .reshape(2048 * 2048 * 2))
    lv = (tabLV.reshape(1024, 8, 128, 4).transpose(0, 1, 3, 2)
          .reshape(1024 * 1024 * 4))
    nhl = (tabNHL.reshape(1024, 8, 128, 4).transpose(0, 1, 3, 2)
           .reshape(1024 * 1024 * 4))
    rad = (tabRADec.transpose(2, 0, 1).reshape(3, 128, 8, 8, 128)
           .transpose(0, 1, 3, 2, 4).reshape(3 * 1024 * 1024))
    lvr = tabLVR.reshape(2048 * 2048)

    x, y = _sc_kernel(
        nv_in(roughNoh), nv_in(anisoToh), nv_in(nDotLV), nv_in(metalLoh),
        nv_in(subCg), nv_in(spst), nv_in(shst), nv_in(lumCs),
        raenc,
        tabCgEnc.reshape(2048),
        tabRSEnc.reshape(64 * 64, 3),
        rad,
        tabCgDec,
        lvr,
        lv,
        nhl,
    )
    return (x[:, None], y[:, None])
